# baseline (speedup 1.0000x reference)
import functools

import jax
import jax.numpy as jnp
from jax import lax
from jax.experimental import pallas as pl
from jax.experimental.pallas import tpu as pltpu
from jax.experimental.pallas import tpu_sc as plsc

_W = 256
_IP = 2048


def _sc_gather(table, idx):
    n = idx.shape[0]
    d = table.shape[1]
    mesh = plsc.VectorSubcoreMesh(core_axis_name="c", subcore_axis_name="s")

    @functools.partial(
        pl.kernel,
        out_type=jax.ShapeDtypeStruct((n, d), table.dtype),
        mesh=mesh,
        scratch_types=[pltpu.VMEM_SHARED(table.shape, table.dtype)],
        compiler_params=pltpu.CompilerParams(use_tc_tiling_on_sc=False),
    )
    def k(table_hbm, i_hbm, o_hbm, table_s):
        @pl.when(lax.axis_index("s") == 0)
        def _():
            pltpu.sync_copy(table_hbm, table_s)

        plsc.subcore_barrier()

        def body(i_vmem, o_vmem):
            pltpu.sync_copy(table_s.at[i_vmem], o_vmem)

        pltpu.emit_pipeline(
            body,
            grid=(n // _W,),
            in_specs=[pl.BlockSpec((_W,), lambda i: (i,))],
            out_specs=[pl.BlockSpec((_W, d), lambda i: (i, 0))],
            core_axis_name=("c", "s"),
            dimension_semantics=(pltpu.PARALLEL,),
        )(i_hbm, o_hbm)

    return k(table, idx)


def _tc_transpose(rows, b, s, d):
    h = b // 2
    y = rows.reshape(-1).reshape(s, h, 2 * d)

    def body(x_ref, o_ref):
        z = x_ref[0].T
        q0 = pl.program_id(1) * _IP
        o_ref[0, :, pl.ds(q0, _IP)] = z[:d]
        o_ref[0, :, pl.ds(h + q0, _IP)] = z[d:]

    out_t = pl.pallas_call(
        body,
        grid=(s, h // _IP),
        in_specs=[pl.BlockSpec((1, _IP, 2 * d), lambda j, q: (j, q, 0))],
        out_specs=pl.BlockSpec((1, d, b), lambda j, q: (j, 0, 0)),
        out_shape=jax.ShapeDtypeStruct((s, d, b), jnp.float32),
    )(y)
    return out_t.transpose(2, 0, 1)


def kernel(species, elementnum_to_vector):
    b, s = species.shape
    d = elementnum_to_vector.shape[1]
    idx = (
        species.T.reshape(s, 2, b // 2).transpose(0, 2, 1).reshape(b * s)
    )
    rows = _sc_gather(elementnum_to_vector, idx)
    return _tc_transpose(rows, b, s, d)


# speedup vs baseline: 1.2220x; 1.2220x over previous
import functools

import jax
import jax.numpy as jnp
from jax import lax
from jax.experimental import pallas as pl
from jax.experimental.pallas import tpu as pltpu
from jax.experimental.pallas import tpu_sc as plsc

_W = 256
_I = 4096


def _sc_gather(table_pad, idx):
    n = idx.shape[0]
    dp = table_pad.shape[1]
    mesh = plsc.VectorSubcoreMesh(core_axis_name="c", subcore_axis_name="s")

    @functools.partial(
        pl.kernel,
        out_type=jax.ShapeDtypeStruct((n, dp), table_pad.dtype),
        mesh=mesh,
        scratch_types=[pltpu.VMEM_SHARED(table_pad.shape, table_pad.dtype)],
        compiler_params=pltpu.CompilerParams(use_tc_tiling_on_sc=False),
    )
    def k(table_hbm, i_hbm, o_hbm, table_s):
        @pl.when(lax.axis_index("s") == 0)
        def _():
            pltpu.sync_copy(table_hbm, table_s)

        plsc.subcore_barrier()

        def body(i_vmem, o_vmem):
            pltpu.sync_copy(table_s.at[i_vmem], o_vmem)

        pltpu.emit_pipeline(
            body,
            grid=(n // _W,),
            in_specs=[pl.BlockSpec((_W,), lambda i: (i,))],
            out_specs=[pl.BlockSpec((_W, dp), lambda i: (i, 0))],
            core_axis_name=("c", "s"),
            dimension_semantics=(pltpu.PARALLEL,),
        )(i_hbm, o_hbm)

    return k(table_pad, idx)


def _tc_transpose(rows, b, s, d):
    y = rows.reshape(-1).reshape(s, b, 128)

    def body(x_ref, o_ref):
        o_ref[0] = x_ref[0].T[:d, :]

    out_t = pl.pallas_call(
        body,
        grid=(s, b // _I),
        in_specs=[pl.BlockSpec((1, _I, 128), lambda j, i: (j, i, 0))],
        out_specs=pl.BlockSpec((1, d, _I), lambda j, i: (j, 0, i)),
        out_shape=jax.ShapeDtypeStruct((s, d, b), jnp.float32),
    )(y)
    return out_t.transpose(2, 0, 1)


def kernel(species, elementnum_to_vector):
    b, s = species.shape
    d = elementnum_to_vector.shape[1]
    table_pad = jnp.pad(elementnum_to_vector, ((0, 0), (0, 128 - d)))
    idx = species.T.reshape(b * s)
    rows = _sc_gather(table_pad, idx)
    return _tc_transpose(rows, b, s, d)
